# baseline (speedup 1.0000x reference)
import functools

import jax
import jax.numpy as jnp
from jax import lax
from jax.experimental import pallas as pl
from jax.experimental.pallas import tpu as pltpu
from jax.experimental.pallas import tpu_sc as plsc

_D = 1024
_NC = 2
_NS = 16
_NW = _NC * _NS
_L = 16


@functools.cache
def _build(b_total, vocab):
    rows_per_w = b_total // _NW
    ngroup = rows_per_w // _L
    mesh = plsc.VectorSubcoreMesh(core_axis_name="c", subcore_axis_name="s")

    @functools.partial(
        pl.kernel,
        mesh=mesh,
        out_type=jax.ShapeDtypeStruct((b_total, _D), jnp.float32),
        scratch_types=[
            pltpu.VMEM((rows_per_w,), jnp.int32),
            pltpu.VMEM((vocab, _D), jnp.float32),
            pltpu.SemaphoreType.DMA,
            pltpu.SemaphoreType.DMA,
        ],
    )
    def k(table_hbm, idx_hbm, out_hbm, idx_v, table_v, s0, s1):
        wid = lax.axis_index("s") * _NC + lax.axis_index("c")
        base = wid * rows_per_w
        q8, r8 = divmod(vocab // 8, 4)
        sizes = [8 * (q8 + (1 if i < r8 else 0)) for i in range(4)]
        offs = [sum(sizes[:i]) for i in range(4)]
        pieces = tuple(zip(offs, sizes))
        wm = lax.rem(wid, 4)
        for m in range(4):
            @pl.when(wm == m)
            def _(m=m):
                for q in range(4):
                    st, ln = pieces[(m + q) % 4]
                    pltpu.async_copy(
                        table_hbm.at[pl.ds(st, ln)], table_v.at[pl.ds(st, ln)],
                        s0)
        pltpu.async_copy(idx_hbm.at[pl.ds(base, rows_per_w)], idx_v, s1)
        for m in range(4):
            @pl.when(wm == m)
            def _(m=m):
                for q in range(4):
                    st, ln = pieces[(m + q) % 4]
                    pltpu.make_async_copy(
                        table_hbm.at[pl.ds(st, ln)], table_v.at[pl.ds(st, ln)],
                        s0).wait()
        pltpu.make_async_copy(
            idx_hbm.at[pl.ds(base, rows_per_w)], idx_v, s1).wait()
        ssem = (s0, s1)

        def extract(t):
            v = idx_v[pl.ds(t * _L, _L)]
            return tuple(
                jnp.squeeze(lax.slice(v, (r,), (r + 1,))) for r in range(_L))

        def emit_half(rows, half, t):
            for r in range(8 * half, 8 * half + 8):
                pltpu.async_copy(
                    table_v.at[rows[r]],
                    out_hbm.at[base + t * _L + r],
                    ssem[half])

        def drain_half(half):
            pltpu.make_async_copy(
                table_v.at[pl.ds(0, 8)],
                out_hbm.at[pl.ds(base, 8)], ssem[half]).wait()

        rows0 = extract(0)
        emit_half(rows0, 0, 0)
        emit_half(rows0, 1, 0)

        def body(t, rows):
            nxt = extract(t + 1)
            drain_half(0)
            emit_half(rows, 0, t)
            drain_half(1)
            emit_half(rows, 1, t)
            return nxt

        last = lax.fori_loop(1, ngroup - 1, body, extract(1))
        drain_half(0)
        emit_half(last, 0, ngroup - 1)
        drain_half(1)
        emit_half(last, 1, ngroup - 1)
        drain_half(0)
        drain_half(1)

    return k


def kernel(lang_ids, embeddings):
    b, s = lang_ids.shape
    idx = lang_ids.reshape(-1)
    pad = (-embeddings.shape[0]) % 8
    emb = jnp.pad(embeddings, ((0, pad), (0, 0)))
    out = _build(b * s, emb.shape[0])(emb, idx)
    return out.reshape(b, s, _D)


# speedup vs baseline: 1.0169x; 1.0169x over previous
import functools

import jax
import jax.numpy as jnp
from jax import lax
from jax.experimental import pallas as pl
from jax.experimental.pallas import tpu as pltpu
from jax.experimental.pallas import tpu_sc as plsc

_D = 1024
_NC = 2
_NS = 16
_NW = _NC * _NS
_L = 16


@functools.cache
def _build(b_total, vocab):
    rows_per_w = b_total // _NW
    ngroup = rows_per_w // _L
    mesh = plsc.VectorSubcoreMesh(core_axis_name="c", subcore_axis_name="s")

    @functools.partial(
        pl.kernel,
        mesh=mesh,
        out_type=jax.ShapeDtypeStruct((b_total, _D), jnp.float32),
        scratch_types=[
            pltpu.VMEM((rows_per_w,), jnp.int32),
            pltpu.VMEM((vocab, _D), jnp.float32),
            pltpu.SemaphoreType.DMA,
            pltpu.SemaphoreType.DMA,
        ],
    )
    def k(table_hbm, idx_hbm, out_hbm, idx_v, table_v, s0, s1):
        wid = lax.axis_index("s") * _NC + lax.axis_index("c")
        base = wid * rows_per_w
        q8, r8 = divmod(vocab // 8, 4)
        sizes = [8 * (q8 + (1 if i < r8 else 0)) for i in range(4)]
        offs = [sum(sizes[:i]) for i in range(4)]
        pieces = tuple(zip(offs, sizes))
        wm = lax.rem(wid, 4)
        for m in range(4):
            @pl.when(wm == m)
            def _(m=m):
                for q in range(4):
                    st, ln = pieces[(m + q) % 4]
                    pltpu.async_copy(
                        table_hbm.at[pl.ds(st, ln)], table_v.at[pl.ds(st, ln)],
                        s0)
        pltpu.async_copy(idx_hbm.at[pl.ds(base, rows_per_w)], idx_v, s1)
        for m in range(4):
            @pl.when(wm == m)
            def _(m=m):
                for q in range(4):
                    st, ln = pieces[(m + q) % 4]
                    pltpu.make_async_copy(
                        table_hbm.at[pl.ds(st, ln)], table_v.at[pl.ds(st, ln)],
                        s0).wait()
        pltpu.make_async_copy(
            idx_hbm.at[pl.ds(base, rows_per_w)], idx_v, s1).wait()
        ssem = (s0, s1)

        def emit_half(v, half, t):
            for r in range(8 * half, 8 * half + 8):
                row = jnp.squeeze(lax.slice(v, (r,), (r + 1,)))
                pltpu.async_copy(
                    table_v.at[row],
                    out_hbm.at[base + t * _L + r],
                    ssem[half])

        def drain_half(half):
            pltpu.make_async_copy(
                table_v.at[pl.ds(0, 8)],
                out_hbm.at[pl.ds(base, 8)], ssem[half]).wait()

        v0 = idx_v[pl.ds(0, _L)]
        emit_half(v0, 0, 0)
        emit_half(v0, 1, 0)

        def body(t, carry):
            v = idx_v[pl.ds(t * _L, _L)]
            drain_half(0)
            emit_half(v, 0, t)
            drain_half(1)
            emit_half(v, 1, t)
            return carry

        lax.fori_loop(1, ngroup, body, 0)
        drain_half(0)
        drain_half(1)

    return k


def kernel(lang_ids, embeddings):
    b, s = lang_ids.shape
    idx = lang_ids.reshape(-1)
    pad = (-embeddings.shape[0]) % 8
    emb = jnp.pad(embeddings, ((0, pad), (0, 0)))
    out = _build(b * s, emb.shape[0])(emb, idx)
    return out.reshape(b, s, _D)
